# baseline (speedup 1.0000x reference)
import functools

import jax
import jax.numpy as jnp
from jax import lax
from jax.experimental import pallas as pl
from jax.experimental.pallas import tpu as pltpu
from jax.experimental.pallas import tpu_sc as plsc

_NC = 2
_NS = 16
_NW = _NC * _NS
_W = 400


def _gather_sc(table, idx, n, d):
    b_per_w = n // _NW
    n_chunks = b_per_w // _W
    mesh = plsc.VectorSubcoreMesh(core_axis_name="c", subcore_axis_name="s")

    assert n_chunks % 2 == 0 and n_chunks >= 4

    @functools.partial(
        pl.kernel,
        out_type=jax.ShapeDtypeStruct((n, d), jnp.float32),
        mesh=mesh,
        scratch_types=[
            pltpu.VMEM((b_per_w,), jnp.int32),
            pltpu.VMEM((_W, d), jnp.float32),
            pltpu.VMEM((_W, d), jnp.float32),
            pltpu.SemaphoreType.DMA,
            pltpu.SemaphoreType.DMA,
            pltpu.SemaphoreType.DMA,
            pltpu.SemaphoreType.DMA,
        ],
    )
    def gather_kernel(table_hbm, idx_hbm, out_hbm, idx_v, rows0, rows1,
                      gsem0, gsem1, osem0, osem1):
        wid = lax.axis_index("s") * _NC + lax.axis_index("c")
        base = wid * b_per_w
        pltpu.sync_copy(idx_hbm.at[pl.ds(base, b_per_w)], idx_v)

        rows = (rows0, rows1)
        gsem = (gsem0, gsem1)
        osem = (osem0, osem1)

        def gstart(c, b):
            pltpu.async_copy(
                table_hbm.at[idx_v.at[pl.ds(c * _W, _W)]], rows[b], gsem[b]
            )

        def gwait(b):
            pltpu.make_async_copy(
                table_hbm.at[idx_v.at[pl.ds(0, _W)]], rows[b], gsem[b]
            ).wait()

        def ostart(c, b):
            pltpu.async_copy(rows[b], out_hbm.at[pl.ds(base + c * _W, _W)],
                             osem[b])

        def owait(b):
            pltpu.make_async_copy(rows[b], out_hbm.at[pl.ds(base, _W)],
                                  osem[b]).wait()

        gstart(0, 0)
        gstart(1, 1)
        gwait(0)
        ostart(0, 0)

        @pl.loop(1, n_chunks - 1, step=2)
        def _(c):
            owait(0)
            gstart(c + 1, 0)
            gwait(1)
            ostart(c, 1)
            owait(1)
            gstart(c + 2, 1)
            gwait(0)
            ostart(c + 1, 0)

        owait(0)
        gwait(1)
        ostart(n_chunks - 1, 1)
        owait(1)

    return gather_kernel(table, idx)


def kernel(input_ids, base_weight, new_weight):
    b, h = input_ids.shape
    d = base_weight.shape[1]
    table = jnp.concatenate([base_weight, new_weight], axis=0)
    idx = input_ids.reshape(-1).astype(jnp.int32)
    out = _gather_sc(table, idx, idx.shape[0], d)
    return out.reshape(b, h, d)
